# baseline (speedup 1.0000x reference)
import functools
import math

import jax
import jax.numpy as jnp
from jax.experimental import pallas as pl
from jax.experimental.pallas import tpu as pltpu

_E = 64
_C = 32
_D = 1024
_F = 4096
_BF = 2048
_NF = _F // _BF


def _ffn_kernel(x_ref, w0_ref, w1_ref, o_ref):
    f = pl.program_id(1)
    x = x_ref[0, 0]
    w0 = w0_ref[0, pl.ds(f * _BF, _BF), :]
    h = jax.lax.dot_general(
        x, w0, (((1,), (1,)), ((), ())),
        preferred_element_type=jnp.float32,
        precision=jax.lax.Precision.DEFAULT,
    )
    h = 0.5 * h * (1.0 + jax.lax.erf(h * (1.0 / math.sqrt(2.0))))
    w1 = w1_ref[0]
    part = jax.lax.dot_general(
        h, w1, (((1,), (1,)), ((), ())),
        preferred_element_type=jnp.float32,
        precision=jax.lax.Precision.DEFAULT,
    )

    @pl.when(f == 0)
    def _init():
        o_ref[0, 0] = part

    @pl.when(f != 0)
    def _acc():
        o_ref[0, 0] += part


@functools.partial(jax.jit, static_argnames=())
def _run(inputs, W0, W1):
    g = inputs.shape[0]
    out = pl.pallas_call(
        _ffn_kernel,
        grid=(_E, _NF),
        in_specs=[
            pl.BlockSpec((1, 1, _C, _D), lambda e, f: (0, e, 0, 0)),
            pl.BlockSpec((1, _F, _D), lambda e, f: (e, 0, 0)),
            pl.BlockSpec((1, _D, _BF), lambda e, f: (e, 0, f)),
        ],
        out_specs=pl.BlockSpec((1, 1, _C, _D), lambda e, f: (0, e, 0, 0)),
        out_shape=jax.ShapeDtypeStruct((g, _E, _C, _D), jnp.float32),
        compiler_params=pltpu.CompilerParams(
            dimension_semantics=("parallel", "arbitrary"),
        ),
    )(inputs, W0, W1)
    return out


def kernel(output_tensor, inputs, W0, W1):
    return _run(inputs, W0, W1)


# speedup vs baseline: 1.1201x; 1.1201x over previous
import functools
import math

import jax
import jax.numpy as jnp
from jax.experimental import pallas as pl
from jax.experimental.pallas import tpu as pltpu

_E = 64
_C = 32
_D = 1024
_F = 4096
_BF = 1024
_NF = _F // _BF


def _mm(a, b):
    return jax.lax.dot_general(
        a, b, (((1,), (1,)), ((), ())),
        preferred_element_type=jnp.float32,
        precision=jax.lax.Precision.DEFAULT,
    )


def _ffn_kernel(x_ref, w0_ref, w1_ref, o_ref):
    f = pl.program_id(1)
    parts = []
    for i in range(2):
        x = x_ref[0, i]
        h = _mm(x, w0_ref[i])
        h = 0.5 * h * (1.0 + jax.lax.erf(h * (1.0 / math.sqrt(2.0))))
        parts.append(_mm(h, w1_ref[i]))
    part = jnp.stack(parts)[None]

    @pl.when(f == 0)
    def _init():
        o_ref[...] = part

    @pl.when(f != 0)
    def _acc():
        o_ref[...] += part


@functools.partial(jax.jit, static_argnames=())
def _run(inputs, W0, W1):
    g = inputs.shape[0]
    out = pl.pallas_call(
        _ffn_kernel,
        grid=(_E // 2, _NF),
        in_specs=[
            pl.BlockSpec((1, 2, _C, _D), lambda e, f: (0, e, 0, 0)),
            pl.BlockSpec((2, _BF, _D), lambda e, f: (e, f, 0)),
            pl.BlockSpec((2, _D, _BF), lambda e, f: (e, 0, f)),
        ],
        out_specs=pl.BlockSpec((1, 2, _C, _D), lambda e, f: (0, e, 0, 0)),
        out_shape=jax.ShapeDtypeStruct((g, _E, _C, _D), jnp.float32),
        compiler_params=pltpu.CompilerParams(
            dimension_semantics=("parallel", "arbitrary"),
        ),
    )(inputs, W0, W1)
    return out


def kernel(output_tensor, inputs, W0, W1):
    return _run(inputs, W0, W1)


# speedup vs baseline: 1.1224x; 1.0021x over previous
import functools
import math

import jax
import jax.numpy as jnp
from jax.experimental import pallas as pl
from jax.experimental.pallas import tpu as pltpu

_E = 64
_C = 32
_D = 1024
_F = 4096
_BF = 2048
_NF = _F // _BF


def _ffn_kernel(x_ref, w0_ref, w1_ref, o_ref):
    f = pl.program_id(1)
    x = x_ref[0, 0]
    w0 = w0_ref[0]
    h = jax.lax.dot_general(
        x, w0, (((1,), (1,)), ((), ())),
        preferred_element_type=jnp.float32,
        precision=jax.lax.Precision.DEFAULT,
    )
    h = 0.5 * h * (1.0 + jax.lax.erf(h * (1.0 / math.sqrt(2.0))))
    w1 = w1_ref[0]
    part = jax.lax.dot_general(
        h, w1, (((1,), (1,)), ((), ())),
        preferred_element_type=jnp.float32,
        precision=jax.lax.Precision.DEFAULT,
    )

    @pl.when(f == 0)
    def _init():
        o_ref[0, 0] = part

    @pl.when(f != 0)
    def _acc():
        o_ref[0, 0] += part


@functools.partial(jax.jit, static_argnames=())
def _run(inputs, W0, W1):
    g = inputs.shape[0]
    out = pl.pallas_call(
        _ffn_kernel,
        grid=(_E, _NF),
        in_specs=[
            pl.BlockSpec((1, 1, _C, _D), lambda e, f: (0, e, 0, 0)),
            pl.BlockSpec((1, _BF, _D), lambda e, f: (e, f, 0)),
            pl.BlockSpec((1, _D, _BF), lambda e, f: (e, 0, f)),
        ],
        out_specs=pl.BlockSpec((1, 1, _C, _D), lambda e, f: (0, e, 0, 0)),
        out_shape=jax.ShapeDtypeStruct((g, _E, _C, _D), jnp.float32),
        compiler_params=pltpu.CompilerParams(
            dimension_semantics=("parallel", "arbitrary"),
        ),
    )(inputs, W0, W1)
    return out


def kernel(output_tensor, inputs, W0, W1):
    return _run(inputs, W0, W1)
